# baseline (speedup 1.0000x reference)
import jax
import jax.numpy as jnp
from jax.experimental import pallas as pl


def _fc_kernel(p_ref, w_ref, b_ref, o_ref):
    o_ref[...] = jnp.dot(p_ref[...], w_ref[...],
                         preferred_element_type=jnp.float32) + b_ref[...]


def _cheb(x, src, dst, W, b):
    n = x.shape[0]
    deg = jnp.zeros((n,), dtype=x.dtype).at[src].add(1.0)
    deg_inv = jnp.where(deg > 0, 1.0 / deg, 0.0)
    norm = -deg_inv[src]

    def mv(v):
        return jax.ops.segment_sum(norm[:, None] * v[src], dst, num_segments=n)

    Tx0 = x
    out = Tx0 @ W[0]
    Tx1 = mv(x)
    out = out + Tx1 @ W[1]
    for k in range(2, W.shape[0]):
        Tx2 = 2.0 * mv(Tx1) - Tx0
        out = out + Tx2 @ W[k]
        Tx0, Tx1 = Tx1, Tx2
    return out + b


def kernel(x, edge_index, batch, W1, b1, W2, b2, Wfc, bfc):
    src, dst = edge_index[0], edge_index[1]
    h = jax.nn.selu(_cheb(x, src, dst, W1, b1))
    h = jax.nn.selu(_cheb(h, src, dst, W2, b2))
    pooled = jax.ops.segment_sum(h, batch, num_segments=64)
    out = pl.pallas_call(
        _fc_kernel,
        out_shape=jax.ShapeDtypeStruct((64, Wfc.shape[1]), jnp.float32),
    )(pooled, Wfc, bfc)
    return out


# speedup vs baseline: 8.6233x; 8.6233x over previous
import jax
import jax.numpy as jnp
from jax import lax
from jax.experimental import pallas as pl
from jax.experimental.pallas import tpu as pltpu
from jax.experimental.pallas import tpu_sc as plsc

_N = 10000
_E = 320000
_D = 128
_G = 64
_NC, _NS = 2, 16
_CH = 128
_CHUNKS = 80
_EPAD = _CHUNKS * _NC * _NS * _CH
_NA = 10240
_ZR = 80
_RPT = _NA // _NS
_B = 1000
_GRID = _N // _B

_SELU_L = 1.0507009873554805
_SELU_A = 1.6732632423543772


def _selu(v):
    return _SELU_L * jnp.where(v > 0, v, _SELU_A * (jnp.exp(jnp.minimum(v, 0.0)) - 1.0))


def _fill(ref, nrows, value):
    @pl.loop(0, nrows)
    def _row(r):
        for c4 in range(_D // 16):
            ref[r, pl.ds(c4 * 16, 16)] = jnp.full((16,), value, jnp.float32)


def _zero_acc(acc, zbuf, s):
    for b in range(_RPT // _ZR):
        pltpu.sync_copy(zbuf, acc.at[pl.ds(s * _RPT + b * _ZR, _ZR)])


def _acc_out(acc, zbuf, s, yref):
    for b in range(_RPT // _ZR):
        off = s * _RPT + b * _ZR
        pltpu.sync_copy(acc.at[pl.ds(off, _ZR)], zbuf)
        pltpu.sync_copy(zbuf, yref.at[pl.ds(off, _ZR)])


def _mv_body(table_hbm, g_hbm, s_hbm, y0_hbm, y1_hbm,
             acc, zbuf, gall, sall, rows, sem):
    c = lax.axis_index("c")
    s = lax.axis_index("s")
    w = c * _NS + s

    _fill(zbuf, _ZR, 0.0)
    _zero_acc(acc, zbuf, s)
    pltpu.sync_copy(g_hbm.at[pl.ds(w * _CHUNKS, _CHUNKS)], gall)
    pltpu.sync_copy(s_hbm.at[pl.ds(w * _CHUNKS, _CHUNKS)], sall)
    plsc.subcore_barrier()

    @pl.loop(0, _CHUNKS)
    def _chunk(i):
        pltpu.async_copy(table_hbm.at[gall.at[i]], rows, sem).wait()
        pltpu.sync_copy(rows, acc.at[sall.at[i]], add=True)

    plsc.subcore_barrier()

    @pl.when(c == 0)
    def _():
        _acc_out(acc, zbuf, s, y0_hbm)

    @pl.when(c == 1)
    def _():
        _acc_out(acc, zbuf, s, y1_hbm)


def _mv(table, g2, s2):
    mesh = plsc.VectorSubcoreMesh(core_axis_name="c", subcore_axis_name="s")
    f = pl.kernel(
        _mv_body,
        out_type=(jax.ShapeDtypeStruct((_NA, _D), jnp.float32),
                  jax.ShapeDtypeStruct((_NA, _D), jnp.float32)),
        mesh=mesh,
        scratch_types=[
            pltpu.VMEM_SHARED((_NA, _D), jnp.float32),
            pltpu.VMEM((_ZR, _D), jnp.float32),
            pltpu.VMEM((_CHUNKS, _CH), jnp.int32),
            pltpu.VMEM((_CHUNKS, _CH), jnp.int32),
            pltpu.VMEM((_CH, _D), jnp.float32),
            pltpu.SemaphoreType.DMA,
        ],
    )
    return f(table, g2, s2)


def _hist_body(s_hbm, d0_hbm, d1_hbm, acc, zbuf, ones, sall):
    c = lax.axis_index("c")
    s = lax.axis_index("s")
    w = c * _NS + s

    _fill(zbuf, _ZR, 0.0)
    _fill(ones, _CH, 1.0)
    _zero_acc(acc, zbuf, s)
    pltpu.sync_copy(s_hbm.at[pl.ds(w * _CHUNKS, _CHUNKS)], sall)
    plsc.subcore_barrier()

    @pl.loop(0, _CHUNKS)
    def _chunk(i):
        pltpu.sync_copy(ones, acc.at[sall.at[i]], add=True)

    plsc.subcore_barrier()

    @pl.when(c == 0)
    def _():
        _acc_out(acc, zbuf, s, d0_hbm)

    @pl.when(c == 1)
    def _():
        _acc_out(acc, zbuf, s, d1_hbm)


def _hist(s2):
    mesh = plsc.VectorSubcoreMesh(core_axis_name="c", subcore_axis_name="s")
    f = pl.kernel(
        _hist_body,
        out_type=(jax.ShapeDtypeStruct((_NA, _D), jnp.float32),
                  jax.ShapeDtypeStruct((_NA, _D), jnp.float32)),
        mesh=mesh,
        scratch_types=[
            pltpu.VMEM_SHARED((_NA, _D), jnp.float32),
            pltpu.VMEM((_ZR, _D), jnp.float32),
            pltpu.VMEM((_CH, _D), jnp.float32),
            pltpu.VMEM((_CHUNKS, _CH), jnp.int32),
        ],
    )
    return f(s2)


def _row_spec(w):
    return pl.BlockSpec((_B, w), lambda i: (i, 0))


def _full_spec(shape):
    return pl.BlockSpec(shape, lambda i: (0, 0))


def _prep_kernel(x_ref, d0_ref, d1_ref, w0_ref, scale_ref, xs_ref, o_ref):
    deg = d0_ref[:, 0:1] + d1_ref[:, 0:1]
    sc = jnp.where(deg > 0, -1.0 / deg, 0.0)
    scale_ref[...] = sc
    xs_ref[...] = x_ref[...] * sc
    o_ref[...] = jnp.dot(x_ref[...], w0_ref[...], preferred_element_type=jnp.float32)


def _prep(x, d0, d1, w0):
    return pl.pallas_call(
        _prep_kernel,
        grid=(_GRID,),
        in_specs=[_row_spec(_D), _row_spec(_D), _row_spec(_D), _full_spec((_D, _D))],
        out_specs=[_row_spec(1), _row_spec(_D), _row_spec(_D)],
        out_shape=[jax.ShapeDtypeStruct((_N, 1), jnp.float32),
                   jax.ShapeDtypeStruct((_N, _D), jnp.float32),
                   jax.ShapeDtypeStruct((_N, _D), jnp.float32)],
    )(x, d0, d1, w0)


def _step_first_kernel(y0_ref, y1_ref, o_ref, w_ref, scale_ref,
                       t_ref, oo_ref, xs_ref):
    t = y0_ref[...] + y1_ref[...]
    t_ref[...] = t
    oo_ref[...] = o_ref[...] + jnp.dot(t, w_ref[...], preferred_element_type=jnp.float32)
    xs_ref[...] = t * scale_ref[...]


def _step_first(y0, y1, o, w, scale):
    return pl.pallas_call(
        _step_first_kernel,
        grid=(_GRID,),
        in_specs=[_row_spec(_D), _row_spec(_D), _row_spec(_D),
                  _full_spec((_D, _D)), _row_spec(1)],
        out_specs=[_row_spec(_D), _row_spec(_D), _row_spec(_D)],
        out_shape=[jax.ShapeDtypeStruct((_N, _D), jnp.float32),
                   jax.ShapeDtypeStruct((_N, _D), jnp.float32),
                   jax.ShapeDtypeStruct((_N, _D), jnp.float32)],
    )(y0, y1, o, w, scale)


def _step_mid_kernel(y0_ref, y1_ref, tpp_ref, o_ref, w_ref, scale_ref,
                     t_ref, oo_ref, xs_ref):
    t = 2.0 * (y0_ref[...] + y1_ref[...]) - tpp_ref[...]
    t_ref[...] = t
    oo_ref[...] = o_ref[...] + jnp.dot(t, w_ref[...], preferred_element_type=jnp.float32)
    xs_ref[...] = t * scale_ref[...]


def _step_mid(y0, y1, tpp, o, w, scale):
    return pl.pallas_call(
        _step_mid_kernel,
        grid=(_GRID,),
        in_specs=[_row_spec(_D), _row_spec(_D), _row_spec(_D), _row_spec(_D),
                  _full_spec((_D, _D)), _row_spec(1)],
        out_specs=[_row_spec(_D), _row_spec(_D), _row_spec(_D)],
        out_shape=[jax.ShapeDtypeStruct((_N, _D), jnp.float32),
                   jax.ShapeDtypeStruct((_N, _D), jnp.float32),
                   jax.ShapeDtypeStruct((_N, _D), jnp.float32)],
    )(y0, y1, tpp, o, w, scale)


def _step_last_kernel(y0_ref, y1_ref, tpp_ref, o_ref, w_ref, oo_ref):
    t = 2.0 * (y0_ref[...] + y1_ref[...]) - tpp_ref[...]
    oo_ref[...] = o_ref[...] + jnp.dot(t, w_ref[...], preferred_element_type=jnp.float32)


def _step_last(y0, y1, tpp, o, w):
    return pl.pallas_call(
        _step_last_kernel,
        grid=(_GRID,),
        in_specs=[_row_spec(_D), _row_spec(_D), _row_spec(_D), _row_spec(_D),
                  _full_spec((_D, _D))],
        out_specs=_row_spec(_D),
        out_shape=jax.ShapeDtypeStruct((_N, _D), jnp.float32),
    )(y0, y1, tpp, o, w)


def _bridge_kernel(o_ref, b_ref, w_ref, scale_ref, h_ref, o2_ref, xs_ref):
    h = _selu(o_ref[...] + b_ref[...])
    h_ref[...] = h
    o2_ref[...] = jnp.dot(h, w_ref[...], preferred_element_type=jnp.float32)
    xs_ref[...] = h * scale_ref[...]


def _bridge(o, b1r, w20, scale):
    return pl.pallas_call(
        _bridge_kernel,
        grid=(_GRID,),
        in_specs=[_row_spec(_D), _full_spec((1, _D)), _full_spec((_D, _D)),
                  _row_spec(1)],
        out_specs=[_row_spec(_D), _row_spec(_D), _row_spec(_D)],
        out_shape=[jax.ShapeDtypeStruct((_N, _D), jnp.float32),
                   jax.ShapeDtypeStruct((_N, _D), jnp.float32),
                   jax.ShapeDtypeStruct((_N, _D), jnp.float32)],
    )(o, b1r, w20, scale)


def _final_kernel(y0_ref, y1_ref, tpp_ref, o_ref, w_ref, b_ref, batch_ref,
                  wfc_ref, bfc_ref, out_ref, pooled):
    i = pl.program_id(0)
    t = 2.0 * (y0_ref[...] + y1_ref[...]) - tpp_ref[...]
    o = o_ref[...] + jnp.dot(t, w_ref[...], preferred_element_type=jnp.float32)
    h = _selu(o + b_ref[...])
    bids = batch_ref[0, 0, :]
    onehot = (bids[:, None] == lax.broadcasted_iota(jnp.int32, (1, _G), 1)
              ).astype(jnp.float32)
    p = lax.dot_general(onehot, h, (((0,), (0,)), ((), ())),
                        preferred_element_type=jnp.float32)

    @pl.when(i == 0)
    def _():
        pooled[...] = p

    @pl.when(i > 0)
    def _():
        pooled[...] += p

    @pl.when(i == _GRID - 1)
    def _():
        out_ref[...] = jnp.dot(pooled[...], wfc_ref[...],
                               preferred_element_type=jnp.float32) + bfc_ref[...]


def _final(y0, y1, tpp, o, w, b2r, batch3, wfc, bfcr):
    return pl.pallas_call(
        _final_kernel,
        grid=(_GRID,),
        in_specs=[_row_spec(_D), _row_spec(_D), _row_spec(_D), _row_spec(_D),
                  _full_spec((_D, _D)), _full_spec((1, _D)),
                  pl.BlockSpec((1, 1, _B), lambda i: (i, 0, 0)),
                  _full_spec((_D, _G)), _full_spec((1, _G))],
        out_specs=_full_spec((_G, _G)),
        out_shape=jax.ShapeDtypeStruct((_G, _G), jnp.float32),
        scratch_shapes=[pltpu.VMEM((_G, _D), jnp.float32)],
    )(y0, y1, tpp, o, w, b2r, batch3, wfc, bfcr)


def kernel(x, edge_index, batch, W1, b1, W2, b2, Wfc, bfc):
    src, dst = edge_index[0], edge_index[1]

    pad = _EPAD - _E
    ap = jnp.arange(pad, dtype=jnp.int32)
    gather_pad = (ap * 97) % _N
    sink_pad = _N + (ap % 16)
    g2 = jnp.concatenate([src, gather_pad]).reshape(_EPAD // _CH, _CH)
    s2 = jnp.concatenate([dst, sink_pad]).reshape(_EPAD // _CH, _CH)
    hs2 = jnp.concatenate([src, sink_pad]).reshape(_EPAD // _CH, _CH)

    b1r = b1.reshape(1, _D)
    b2r = b2.reshape(1, _D)
    bfcr = bfc.reshape(1, _G)
    batch3 = batch.reshape(_GRID, 1, _B)

    d0, d1 = _hist(hs2)
    scale, xs, o = _prep(x, d0, d1, W1[0])

    tpp = x
    y0, y1 = _mv(xs, g2, s2)
    t, o, xs = _step_first(y0, y1, o, W1[1], scale)
    for k in (2, 3):
        y0, y1 = _mv(xs, g2, s2)
        tn, o, xs = _step_mid(y0, y1, tpp, o, W1[k], scale)
        tpp, t = t, tn
    y0, y1 = _mv(xs, g2, s2)
    o = _step_last(y0, y1, tpp, o, W1[4])

    h, o, xs = _bridge(o, b1r, W2[0], scale)
    tpp = h
    y0, y1 = _mv(xs, g2, s2)
    t, o, xs = _step_first(y0, y1, o, W2[1], scale)
    for k in (2, 3):
        y0, y1 = _mv(xs, g2, s2)
        tn, o, xs = _step_mid(y0, y1, tpp, o, W2[k], scale)
        tpp, t = t, tn
    y0, y1 = _mv(xs, g2, s2)
    return _final(y0, y1, tpp, o, W2[4], b2r, batch3, Wfc, bfcr)


# speedup vs baseline: 12.6186x; 1.4633x over previous
import jax
import jax.numpy as jnp
from jax import lax
from jax.experimental import pallas as pl
from jax.experimental.pallas import tpu as pltpu
from jax.experimental.pallas import tpu_sc as plsc

_N = 10000
_E = 320000
_D = 128
_G = 64
_NC, _NS = 2, 16
_CH = 128
_CHUNKS = 80
_EPAD = _CHUNKS * _NC * _NS * _CH
_NA = 10240
_ZR = 80
_ZZ = 16
_RPT = _NA // _NS
_B = 1000
_GRID = _N // _B

_SELU_L = 1.0507009873554805
_SELU_A = 1.6732632423543772


def _selu(v):
    return _SELU_L * jnp.where(v > 0, v, _SELU_A * (jnp.exp(jnp.minimum(v, 0.0)) - 1.0))


def _fill(ref, nrows, value):
    @pl.loop(0, nrows)
    def _row(r):
        for c4 in range(_D // 16):
            ref[r, pl.ds(c4 * 16, 16)] = jnp.full((16,), value, jnp.float32)


def _zero_acc(acc, zbuf, s):
    @pl.loop(0, _RPT // _ZZ)
    def _z(b):
        pltpu.sync_copy(zbuf, acc.at[pl.ds(s * _RPT + b * _ZZ, _ZZ)])


def _acc_out(acc, rows0, s, yref):
    for b in range(_RPT // _CH):
        off = s * _RPT + b * _CH
        pltpu.sync_copy(acc.at[pl.ds(off, _CH)], rows0)
        pltpu.sync_copy(rows0, yref.at[pl.ds(off, _CH)])


def _mv_body(table_hbm, p_hbm, y0_hbm, y1_hbm,
             acc, zbuf, pall, sidx0, didx0, sidx1, didx1,
             rows0, rows1, gs0, gs1):
    c = lax.axis_index("c")
    s = lax.axis_index("s")
    w = c * _NS + s

    _fill(zbuf, _ZZ, 0.0)
    _zero_acc(acc, zbuf, s)
    pltpu.sync_copy(p_hbm.at[pl.ds(w * _CHUNKS, _CHUNKS)], pall)
    plsc.subcore_barrier()

    def _unpack(i, sbuf, dbuf):
        for j in range(_CH // 16):
            v = pall[i, pl.ds(j * 16, 16)]
            sbuf[pl.ds(j * 16, 16)] = lax.shift_right_logical(v, 14)
            dbuf[pl.ds(j * 16, 16)] = lax.bitwise_and(v, 16383)

    _unpack(0, sidx0, didx0)
    pltpu.async_copy(table_hbm.at[sidx0], rows0, gs0)

    @pl.loop(0, _CHUNKS, step=2)
    def _chunk(i):
        _unpack(i + 1, sidx1, didx1)
        pltpu.make_async_copy(table_hbm.at[sidx0], rows0, gs0).wait()
        pltpu.async_copy(table_hbm.at[sidx1], rows1, gs1)
        pltpu.sync_copy(rows0, acc.at[didx0], add=True)

        @pl.when(i + 2 < _CHUNKS)
        def _():
            _unpack(i + 2, sidx0, didx0)
            pltpu.async_copy(table_hbm.at[sidx0], rows0, gs0)

        pltpu.make_async_copy(table_hbm.at[sidx1], rows1, gs1).wait()
        pltpu.sync_copy(rows1, acc.at[didx1], add=True)

    plsc.subcore_barrier()

    @pl.when(c == 0)
    def _():
        _acc_out(acc, rows0, s, y0_hbm)

    @pl.when(c == 1)
    def _():
        _acc_out(acc, rows0, s, y1_hbm)


def _mv(table, p2):
    mesh = plsc.VectorSubcoreMesh(core_axis_name="c", subcore_axis_name="s")
    f = pl.kernel(
        _mv_body,
        out_type=(jax.ShapeDtypeStruct((_NA, _D), jnp.float32),
                  jax.ShapeDtypeStruct((_NA, _D), jnp.float32)),
        mesh=mesh,
        scratch_types=[
            pltpu.VMEM_SHARED((_NA, _D), jnp.float32),
            pltpu.VMEM((_ZZ, _D), jnp.float32),
            pltpu.VMEM((_CHUNKS, _CH), jnp.int32),
            pltpu.VMEM((_CH,), jnp.int32),
            pltpu.VMEM((_CH,), jnp.int32),
            pltpu.VMEM((_CH,), jnp.int32),
            pltpu.VMEM((_CH,), jnp.int32),
            pltpu.VMEM((_CH, _D), jnp.float32),
            pltpu.VMEM((_CH, _D), jnp.float32),
            pltpu.SemaphoreType.DMA,
            pltpu.SemaphoreType.DMA,
        ],
    )
    return f(table, p2)


def _hist_body(s_hbm, d0_hbm, d1_hbm, acc, zbuf, ones, sall):
    c = lax.axis_index("c")
    s = lax.axis_index("s")
    w = c * _NS + s

    _fill(zbuf, _ZZ, 0.0)
    _fill(ones, _CH, 1.0)
    _zero_acc(acc, zbuf, s)
    pltpu.sync_copy(s_hbm.at[pl.ds(w * _CHUNKS, _CHUNKS)], sall)
    plsc.subcore_barrier()

    @pl.loop(0, _CHUNKS)
    def _chunk(i):
        pltpu.sync_copy(ones, acc.at[sall.at[i]], add=True)

    plsc.subcore_barrier()

    @pl.when(c == 0)
    def _():
        _acc_out(acc, ones, s, d0_hbm)

    @pl.when(c == 1)
    def _():
        _acc_out(acc, ones, s, d1_hbm)


def _hist(s2):
    mesh = plsc.VectorSubcoreMesh(core_axis_name="c", subcore_axis_name="s")
    f = pl.kernel(
        _hist_body,
        out_type=(jax.ShapeDtypeStruct((_NA, _D), jnp.float32),
                  jax.ShapeDtypeStruct((_NA, _D), jnp.float32)),
        mesh=mesh,
        scratch_types=[
            pltpu.VMEM_SHARED((_NA, _D), jnp.float32),
            pltpu.VMEM((_ZZ, _D), jnp.float32),
            pltpu.VMEM((_CH, _D), jnp.float32),
            pltpu.VMEM((_CHUNKS, _CH), jnp.int32),
        ],
    )
    return f(s2)


def _row_spec(w):
    return pl.BlockSpec((_B, w), lambda i: (i, 0))


def _full_spec(shape):
    return pl.BlockSpec(shape, lambda i: (0, 0))


def _prep_kernel(x_ref, d0_ref, d1_ref, w0_ref, scale_ref, xs_ref, o_ref):
    deg = d0_ref[:, 0:1] + d1_ref[:, 0:1]
    sc = jnp.where(deg > 0, -1.0 / deg, 0.0)
    scale_ref[...] = sc
    xs_ref[...] = x_ref[...] * sc
    o_ref[...] = jnp.dot(x_ref[...], w0_ref[...], preferred_element_type=jnp.float32)


def _prep(x, d0, d1, w0):
    return pl.pallas_call(
        _prep_kernel,
        grid=(_GRID,),
        in_specs=[_row_spec(_D), _row_spec(_D), _row_spec(_D), _full_spec((_D, _D))],
        out_specs=[_row_spec(1), _row_spec(_D), _row_spec(_D)],
        out_shape=[jax.ShapeDtypeStruct((_N, 1), jnp.float32),
                   jax.ShapeDtypeStruct((_N, _D), jnp.float32),
                   jax.ShapeDtypeStruct((_N, _D), jnp.float32)],
    )(x, d0, d1, w0)


def _step_first_kernel(y0_ref, y1_ref, o_ref, w_ref, scale_ref,
                       t_ref, oo_ref, xs_ref):
    t = y0_ref[...] + y1_ref[...]
    t_ref[...] = t
    oo_ref[...] = o_ref[...] + jnp.dot(t, w_ref[...], preferred_element_type=jnp.float32)
    xs_ref[...] = t * scale_ref[...]


def _step_first(y0, y1, o, w, scale):
    return pl.pallas_call(
        _step_first_kernel,
        grid=(_GRID,),
        in_specs=[_row_spec(_D), _row_spec(_D), _row_spec(_D),
                  _full_spec((_D, _D)), _row_spec(1)],
        out_specs=[_row_spec(_D), _row_spec(_D), _row_spec(_D)],
        out_shape=[jax.ShapeDtypeStruct((_N, _D), jnp.float32),
                   jax.ShapeDtypeStruct((_N, _D), jnp.float32),
                   jax.ShapeDtypeStruct((_N, _D), jnp.float32)],
    )(y0, y1, o, w, scale)


def _step_mid_kernel(y0_ref, y1_ref, tpp_ref, o_ref, w_ref, scale_ref,
                     t_ref, oo_ref, xs_ref):
    t = 2.0 * (y0_ref[...] + y1_ref[...]) - tpp_ref[...]
    t_ref[...] = t
    oo_ref[...] = o_ref[...] + jnp.dot(t, w_ref[...], preferred_element_type=jnp.float32)
    xs_ref[...] = t * scale_ref[...]


def _step_mid(y0, y1, tpp, o, w, scale):
    return pl.pallas_call(
        _step_mid_kernel,
        grid=(_GRID,),
        in_specs=[_row_spec(_D), _row_spec(_D), _row_spec(_D), _row_spec(_D),
                  _full_spec((_D, _D)), _row_spec(1)],
        out_specs=[_row_spec(_D), _row_spec(_D), _row_spec(_D)],
        out_shape=[jax.ShapeDtypeStruct((_N, _D), jnp.float32),
                   jax.ShapeDtypeStruct((_N, _D), jnp.float32),
                   jax.ShapeDtypeStruct((_N, _D), jnp.float32)],
    )(y0, y1, tpp, o, w, scale)


def _step_last_kernel(y0_ref, y1_ref, tpp_ref, o_ref, w_ref, oo_ref):
    t = 2.0 * (y0_ref[...] + y1_ref[...]) - tpp_ref[...]
    oo_ref[...] = o_ref[...] + jnp.dot(t, w_ref[...], preferred_element_type=jnp.float32)


def _step_last(y0, y1, tpp, o, w):
    return pl.pallas_call(
        _step_last_kernel,
        grid=(_GRID,),
        in_specs=[_row_spec(_D), _row_spec(_D), _row_spec(_D), _row_spec(_D),
                  _full_spec((_D, _D))],
        out_specs=_row_spec(_D),
        out_shape=jax.ShapeDtypeStruct((_N, _D), jnp.float32),
    )(y0, y1, tpp, o, w)


def _bridge_kernel(o_ref, b_ref, w_ref, scale_ref, h_ref, o2_ref, xs_ref):
    h = _selu(o_ref[...] + b_ref[...])
    h_ref[...] = h
    o2_ref[...] = jnp.dot(h, w_ref[...], preferred_element_type=jnp.float32)
    xs_ref[...] = h * scale_ref[...]


def _bridge(o, b1r, w20, scale):
    return pl.pallas_call(
        _bridge_kernel,
        grid=(_GRID,),
        in_specs=[_row_spec(_D), _full_spec((1, _D)), _full_spec((_D, _D)),
                  _row_spec(1)],
        out_specs=[_row_spec(_D), _row_spec(_D), _row_spec(_D)],
        out_shape=[jax.ShapeDtypeStruct((_N, _D), jnp.float32),
                   jax.ShapeDtypeStruct((_N, _D), jnp.float32),
                   jax.ShapeDtypeStruct((_N, _D), jnp.float32)],
    )(o, b1r, w20, scale)


def _final_kernel(y0_ref, y1_ref, tpp_ref, o_ref, w_ref, b_ref, batch_ref,
                  wfc_ref, bfc_ref, out_ref, pooled):
    i = pl.program_id(0)
    t = 2.0 * (y0_ref[...] + y1_ref[...]) - tpp_ref[...]
    o = o_ref[...] + jnp.dot(t, w_ref[...], preferred_element_type=jnp.float32)
    h = _selu(o + b_ref[...])
    bids = batch_ref[0, 0, :]
    onehot = (bids[:, None] == lax.broadcasted_iota(jnp.int32, (1, _G), 1)
              ).astype(jnp.float32)
    p = lax.dot_general(onehot, h, (((0,), (0,)), ((), ())),
                        preferred_element_type=jnp.float32)

    @pl.when(i == 0)
    def _():
        pooled[...] = p

    @pl.when(i > 0)
    def _():
        pooled[...] += p

    @pl.when(i == _GRID - 1)
    def _():
        out_ref[...] = jnp.dot(pooled[...], wfc_ref[...],
                               preferred_element_type=jnp.float32) + bfc_ref[...]


def _final(y0, y1, tpp, o, w, b2r, batch3, wfc, bfcr):
    return pl.pallas_call(
        _final_kernel,
        grid=(_GRID,),
        in_specs=[_row_spec(_D), _row_spec(_D), _row_spec(_D), _row_spec(_D),
                  _full_spec((_D, _D)), _full_spec((1, _D)),
                  pl.BlockSpec((1, 1, _B), lambda i: (i, 0, 0)),
                  _full_spec((_D, _G)), _full_spec((1, _G))],
        out_specs=_full_spec((_G, _G)),
        out_shape=jax.ShapeDtypeStruct((_G, _G), jnp.float32),
        scratch_shapes=[pltpu.VMEM((_G, _D), jnp.float32)],
    )(y0, y1, tpp, o, w, b2r, batch3, wfc, bfcr)


def kernel(x, edge_index, batch, W1, b1, W2, b2, Wfc, bfc):
    src, dst = edge_index[0], edge_index[1]

    pad = _EPAD - _E
    ap = jnp.arange(pad, dtype=jnp.int32)
    gather_pad = (ap * 97) % _N
    sink_pad = _N + (ap % 16)
    gp = jnp.concatenate([src, gather_pad])
    sp = jnp.concatenate([dst, sink_pad])
    p2 = ((gp << 14) | sp).reshape(_EPAD // _CH, _CH)
    hs2 = jnp.concatenate([src, sink_pad]).reshape(_EPAD // _CH, _CH)

    b1r = b1.reshape(1, _D)
    b2r = b2.reshape(1, _D)
    bfcr = bfc.reshape(1, _G)
    batch3 = batch.reshape(_GRID, 1, _B)

    d0, d1 = _hist(hs2)
    scale, xs, o = _prep(x, d0, d1, W1[0])

    tpp = x
    y0, y1 = _mv(xs, p2)
    t, o, xs = _step_first(y0, y1, o, W1[1], scale)
    for k in (2, 3):
        y0, y1 = _mv(xs, p2)
        tn, o, xs = _step_mid(y0, y1, tpp, o, W1[k], scale)
        tpp, t = t, tn
    y0, y1 = _mv(xs, p2)
    o = _step_last(y0, y1, tpp, o, W1[4])

    h, o, xs = _bridge(o, b1r, W2[0], scale)
    tpp = h
    y0, y1 = _mv(xs, p2)
    t, o, xs = _step_first(y0, y1, o, W2[1], scale)
    for k in (2, 3):
        y0, y1 = _mv(xs, p2)
        tn, o, xs = _step_mid(y0, y1, tpp, o, W2[k], scale)
        tpp, t = t, tn
    y0, y1 = _mv(xs, p2)
    return _final(y0, y1, tpp, o, W2[4], b2r, batch3, Wfc, bfcr)


# speedup vs baseline: 12.9937x; 1.0297x over previous
import jax
import jax.numpy as jnp
from jax import lax
from jax.experimental import pallas as pl
from jax.experimental.pallas import tpu as pltpu
from jax.experimental.pallas import tpu_sc as plsc

_N = 10000
_E = 320000
_D = 128
_G = 64
_NC, _NS = 2, 16
_CH = 128
_CHUNKS = 80
_EPAD = _CHUNKS * _NC * _NS * _CH
_NA = 10240
_ZR = 80
_ZZ = 40
_HZ = 40
_RPT = _NA // _NS
_B = 1000
_GRID = _N // _B

_SELU_L = 1.0507009873554805
_SELU_A = 1.6732632423543772


def _selu(v):
    return _SELU_L * jnp.where(v > 0, v, _SELU_A * (jnp.exp(jnp.minimum(v, 0.0)) - 1.0))


def _fill(ref, nrows, ncols, value):
    @pl.loop(0, nrows)
    def _row(r):
        for c4 in range(ncols // 16):
            ref[r, pl.ds(c4 * 16, 16)] = jnp.full((16,), value, jnp.float32)


def _zero_acc(acc, zbuf, s):
    @pl.loop(0, _RPT // _ZZ)
    def _z(b):
        pltpu.sync_copy(zbuf, acc.at[pl.ds(s * _RPT + b * _ZZ, _ZZ)])


def _acc_out(acc, s, yref, buf0, buf1, sem0, sem1):
    nb = _RPT // _CH
    bufs, sems = (buf0, buf1), (sem0, sem1)

    def _off(b):
        return s * _RPT + b * _CH

    pltpu.async_copy(acc.at[pl.ds(_off(0), _CH)], buf0, sem0)
    for b in range(nb):
        cur, sem = bufs[b % 2], sems[b % 2]
        pltpu.make_async_copy(acc.at[pl.ds(_off(b), _CH)], cur, sem).wait()
        if b + 1 < nb:
            pltpu.async_copy(acc.at[pl.ds(_off(b + 1), _CH)],
                             bufs[(b + 1) % 2], sems[(b + 1) % 2])
        pltpu.sync_copy(cur, yref.at[pl.ds(_off(b), _CH)])


def _mv_body(table_hbm, p_hbm, y0_hbm, y1_hbm,
             acc, zbuf, pall, sidx0, didx0, sidx1, didx1,
             rows0, rows1, gs0, gs1):
    c = lax.axis_index("c")
    s = lax.axis_index("s")
    w = c * _NS + s

    _fill(zbuf, _ZZ, _D, 0.0)
    pltpu.async_copy(p_hbm.at[pl.ds(w * _CHUNKS, _CHUNKS)], pall, gs1)
    _zero_acc(acc, zbuf, s)
    pltpu.make_async_copy(p_hbm.at[pl.ds(w * _CHUNKS, _CHUNKS)], pall, gs1).wait()
    plsc.subcore_barrier()

    def _unpack(i, sbuf, dbuf):
        for j in range(_CH // 16):
            v = pall[i, pl.ds(j * 16, 16)]
            sbuf[pl.ds(j * 16, 16)] = lax.shift_right_logical(v, 14)
            dbuf[pl.ds(j * 16, 16)] = lax.bitwise_and(v, 16383)

    _unpack(0, sidx0, didx0)
    pltpu.async_copy(table_hbm.at[sidx0], rows0, gs0)

    @pl.loop(0, _CHUNKS, step=2)
    def _chunk(i):
        _unpack(i + 1, sidx1, didx1)
        pltpu.make_async_copy(table_hbm.at[sidx0], rows0, gs0).wait()
        pltpu.async_copy(table_hbm.at[sidx1], rows1, gs1)
        pltpu.sync_copy(rows0, acc.at[didx0], add=True)

        @pl.when(i + 2 < _CHUNKS)
        def _():
            _unpack(i + 2, sidx0, didx0)
            pltpu.async_copy(table_hbm.at[sidx0], rows0, gs0)

        pltpu.make_async_copy(table_hbm.at[sidx1], rows1, gs1).wait()
        pltpu.sync_copy(rows1, acc.at[didx1], add=True)

    plsc.subcore_barrier()

    @pl.when(c == 0)
    def _():
        _acc_out(acc, s, y0_hbm, rows0, rows1, gs0, gs1)

    @pl.when(c == 1)
    def _():
        _acc_out(acc, s, y1_hbm, rows0, rows1, gs0, gs1)


def _mv(table, p2):
    mesh = plsc.VectorSubcoreMesh(core_axis_name="c", subcore_axis_name="s")
    f = pl.kernel(
        _mv_body,
        out_type=(jax.ShapeDtypeStruct((_NA, _D), jnp.float32),
                  jax.ShapeDtypeStruct((_NA, _D), jnp.float32)),
        mesh=mesh,
        scratch_types=[
            pltpu.VMEM_SHARED((_NA, _D), jnp.float32),
            pltpu.VMEM((_ZZ, _D), jnp.float32),
            pltpu.VMEM((_CHUNKS, _CH), jnp.int32),
            pltpu.VMEM((_CH,), jnp.int32),
            pltpu.VMEM((_CH,), jnp.int32),
            pltpu.VMEM((_CH,), jnp.int32),
            pltpu.VMEM((_CH,), jnp.int32),
            pltpu.VMEM((_CH, _D), jnp.float32),
            pltpu.VMEM((_CH, _D), jnp.float32),
            pltpu.SemaphoreType.DMA,
            pltpu.SemaphoreType.DMA,
        ],
    )
    return f(table, p2)


def _hist_body(s_hbm, d0_hbm, d1_hbm, acc, zbuf, ones, sall):
    c = lax.axis_index("c")
    s = lax.axis_index("s")
    w = c * _NS + s

    _fill(zbuf, _HZ, _D, 0.0)
    _fill(ones, _CH, _D, 1.0)

    @pl.loop(0, _RPT // _HZ)
    def _z(b):
        pltpu.sync_copy(zbuf, acc.at[pl.ds(s * _RPT + b * _HZ, _HZ)])

    pltpu.sync_copy(s_hbm.at[pl.ds(w * _CHUNKS, _CHUNKS)], sall)
    plsc.subcore_barrier()

    @pl.loop(0, _CHUNKS)
    def _chunk(i):
        pltpu.sync_copy(ones, acc.at[sall.at[i]], add=True)

    plsc.subcore_barrier()

    def _wout(dref):
        for b in range(_RPT // _CH):
            off = s * _RPT + b * _CH
            pltpu.sync_copy(acc.at[pl.ds(off, _CH)], ones)
            pltpu.sync_copy(ones, dref.at[pl.ds(off, _CH)])

    @pl.when(c == 0)
    def _():
        _wout(d0_hbm)

    @pl.when(c == 1)
    def _():
        _wout(d1_hbm)


def _hist(s2):
    mesh = plsc.VectorSubcoreMesh(core_axis_name="c", subcore_axis_name="s")
    f = pl.kernel(
        _hist_body,
        out_type=(jax.ShapeDtypeStruct((_NA, _D), jnp.float32),
                  jax.ShapeDtypeStruct((_NA, _D), jnp.float32)),
        mesh=mesh,
        scratch_types=[
            pltpu.VMEM_SHARED((_NA, _D), jnp.float32),
            pltpu.VMEM((_HZ, _D), jnp.float32),
            pltpu.VMEM((_CH, _D), jnp.float32),
            pltpu.VMEM((_CHUNKS, _CH), jnp.int32),
        ],
    )
    return f(s2)


def _row_spec(w):
    return pl.BlockSpec((_B, w), lambda i: (i, 0))


def _full_spec(shape):
    return pl.BlockSpec(shape, lambda i: (0, 0))


def _prep_kernel(x_ref, d0_ref, d1_ref, w0_ref, scale_ref, xs_ref, o_ref):
    deg = d0_ref[:, 0:1] + d1_ref[:, 0:1]
    sc = jnp.where(deg > 0, -1.0 / deg, 0.0)
    scale_ref[...] = sc
    xs_ref[...] = x_ref[...] * sc
    o_ref[...] = jnp.dot(x_ref[...], w0_ref[...], preferred_element_type=jnp.float32)


def _prep(x, d0, d1, w0):
    return pl.pallas_call(
        _prep_kernel,
        grid=(_GRID,),
        in_specs=[_row_spec(_D), _row_spec(_D), _row_spec(_D), _full_spec((_D, _D))],
        out_specs=[_row_spec(1), _row_spec(_D), _row_spec(_D)],
        out_shape=[jax.ShapeDtypeStruct((_N, 1), jnp.float32),
                   jax.ShapeDtypeStruct((_N, _D), jnp.float32),
                   jax.ShapeDtypeStruct((_N, _D), jnp.float32)],
    )(x, d0, d1, w0)


def _step_first_kernel(y0_ref, y1_ref, o_ref, w_ref, scale_ref,
                       t_ref, oo_ref, xs_ref):
    t = y0_ref[...] + y1_ref[...]
    t_ref[...] = t
    oo_ref[...] = o_ref[...] + jnp.dot(t, w_ref[...], preferred_element_type=jnp.float32)
    xs_ref[...] = t * scale_ref[...]


def _step_first(y0, y1, o, w, scale):
    return pl.pallas_call(
        _step_first_kernel,
        grid=(_GRID,),
        in_specs=[_row_spec(_D), _row_spec(_D), _row_spec(_D),
                  _full_spec((_D, _D)), _row_spec(1)],
        out_specs=[_row_spec(_D), _row_spec(_D), _row_spec(_D)],
        out_shape=[jax.ShapeDtypeStruct((_N, _D), jnp.float32),
                   jax.ShapeDtypeStruct((_N, _D), jnp.float32),
                   jax.ShapeDtypeStruct((_N, _D), jnp.float32)],
    )(y0, y1, o, w, scale)


def _step_mid_kernel(y0_ref, y1_ref, tpp_ref, o_ref, w_ref, scale_ref,
                     t_ref, oo_ref, xs_ref):
    t = 2.0 * (y0_ref[...] + y1_ref[...]) - tpp_ref[...]
    t_ref[...] = t
    oo_ref[...] = o_ref[...] + jnp.dot(t, w_ref[...], preferred_element_type=jnp.float32)
    xs_ref[...] = t * scale_ref[...]


def _step_mid(y0, y1, tpp, o, w, scale):
    return pl.pallas_call(
        _step_mid_kernel,
        grid=(_GRID,),
        in_specs=[_row_spec(_D), _row_spec(_D), _row_spec(_D), _row_spec(_D),
                  _full_spec((_D, _D)), _row_spec(1)],
        out_specs=[_row_spec(_D), _row_spec(_D), _row_spec(_D)],
        out_shape=[jax.ShapeDtypeStruct((_N, _D), jnp.float32),
                   jax.ShapeDtypeStruct((_N, _D), jnp.float32),
                   jax.ShapeDtypeStruct((_N, _D), jnp.float32)],
    )(y0, y1, tpp, o, w, scale)


def _step_last_kernel(y0_ref, y1_ref, tpp_ref, o_ref, w_ref, oo_ref):
    t = 2.0 * (y0_ref[...] + y1_ref[...]) - tpp_ref[...]
    oo_ref[...] = o_ref[...] + jnp.dot(t, w_ref[...], preferred_element_type=jnp.float32)


def _step_last(y0, y1, tpp, o, w):
    return pl.pallas_call(
        _step_last_kernel,
        grid=(_GRID,),
        in_specs=[_row_spec(_D), _row_spec(_D), _row_spec(_D), _row_spec(_D),
                  _full_spec((_D, _D))],
        out_specs=_row_spec(_D),
        out_shape=jax.ShapeDtypeStruct((_N, _D), jnp.float32),
    )(y0, y1, tpp, o, w)


def _bridge_kernel(o_ref, b_ref, w_ref, scale_ref, h_ref, o2_ref, xs_ref):
    h = _selu(o_ref[...] + b_ref[...])
    h_ref[...] = h
    o2_ref[...] = jnp.dot(h, w_ref[...], preferred_element_type=jnp.float32)
    xs_ref[...] = h * scale_ref[...]


def _bridge(o, b1r, w20, scale):
    return pl.pallas_call(
        _bridge_kernel,
        grid=(_GRID,),
        in_specs=[_row_spec(_D), _full_spec((1, _D)), _full_spec((_D, _D)),
                  _row_spec(1)],
        out_specs=[_row_spec(_D), _row_spec(_D), _row_spec(_D)],
        out_shape=[jax.ShapeDtypeStruct((_N, _D), jnp.float32),
                   jax.ShapeDtypeStruct((_N, _D), jnp.float32),
                   jax.ShapeDtypeStruct((_N, _D), jnp.float32)],
    )(o, b1r, w20, scale)


def _final_kernel(y0_ref, y1_ref, tpp_ref, o_ref, w_ref, b_ref, batch_ref,
                  wfc_ref, bfc_ref, out_ref, pooled):
    i = pl.program_id(0)
    t = 2.0 * (y0_ref[...] + y1_ref[...]) - tpp_ref[...]
    o = o_ref[...] + jnp.dot(t, w_ref[...], preferred_element_type=jnp.float32)
    h = _selu(o + b_ref[...])
    bids = batch_ref[0, 0, :]
    onehot = (bids[:, None] == lax.broadcasted_iota(jnp.int32, (1, _G), 1)
              ).astype(jnp.float32)
    p = lax.dot_general(onehot, h, (((0,), (0,)), ((), ())),
                        preferred_element_type=jnp.float32)

    @pl.when(i == 0)
    def _():
        pooled[...] = p

    @pl.when(i > 0)
    def _():
        pooled[...] += p

    @pl.when(i == _GRID - 1)
    def _():
        out_ref[...] = jnp.dot(pooled[...], wfc_ref[...],
                               preferred_element_type=jnp.float32) + bfc_ref[...]


def _final(y0, y1, tpp, o, w, b2r, batch3, wfc, bfcr):
    return pl.pallas_call(
        _final_kernel,
        grid=(_GRID,),
        in_specs=[_row_spec(_D), _row_spec(_D), _row_spec(_D), _row_spec(_D),
                  _full_spec((_D, _D)), _full_spec((1, _D)),
                  pl.BlockSpec((1, 1, _B), lambda i: (i, 0, 0)),
                  _full_spec((_D, _G)), _full_spec((1, _G))],
        out_specs=_full_spec((_G, _G)),
        out_shape=jax.ShapeDtypeStruct((_G, _G), jnp.float32),
        scratch_shapes=[pltpu.VMEM((_G, _D), jnp.float32)],
    )(y0, y1, tpp, o, w, b2r, batch3, wfc, bfcr)


def kernel(x, edge_index, batch, W1, b1, W2, b2, Wfc, bfc):
    src, dst = edge_index[0], edge_index[1]

    pad = _EPAD - _E
    ap = jnp.arange(pad, dtype=jnp.int32)
    gather_pad = (ap * 97) % _N
    sink_pad = _N + (ap % 16)
    gp = jnp.concatenate([src, gather_pad])
    sp = jnp.concatenate([dst, sink_pad])
    p2 = ((gp << 14) | sp).reshape(_EPAD // _CH, _CH)
    hs2 = jnp.concatenate([src, sink_pad]).reshape(_EPAD // _CH, _CH)

    b1r = b1.reshape(1, _D)
    b2r = b2.reshape(1, _D)
    bfcr = bfc.reshape(1, _G)
    batch3 = batch.reshape(_GRID, 1, _B)

    d0, d1 = _hist(hs2)
    scale, xs, o = _prep(x, d0, d1, W1[0])

    tpp = x
    y0, y1 = _mv(xs, p2)
    t, o, xs = _step_first(y0, y1, o, W1[1], scale)
    for k in (2, 3):
        y0, y1 = _mv(xs, p2)
        tn, o, xs = _step_mid(y0, y1, tpp, o, W1[k], scale)
        tpp, t = t, tn
    y0, y1 = _mv(xs, p2)
    o = _step_last(y0, y1, tpp, o, W1[4])

    h, o, xs = _bridge(o, b1r, W2[0], scale)
    tpp = h
    y0, y1 = _mv(xs, p2)
    t, o, xs = _step_first(y0, y1, o, W2[1], scale)
    for k in (2, 3):
        y0, y1 = _mv(xs, p2)
        tn, o, xs = _step_mid(y0, y1, tpp, o, W2[k], scale)
        tpp, t = t, tn
    y0, y1 = _mv(xs, p2)
    return _final(y0, y1, tpp, o, W2[4], b2r, batch3, Wfc, bfcr)


# speedup vs baseline: 13.0199x; 1.0020x over previous
import jax
import jax.numpy as jnp
from jax import lax
from jax.experimental import pallas as pl
from jax.experimental.pallas import tpu as pltpu
from jax.experimental.pallas import tpu_sc as plsc

_N = 10000
_E = 320000
_D = 128
_G = 64
_NC, _NS = 2, 16
_CH = 128
_CHUNKS = 80
_EPAD = _CHUNKS * _NC * _NS * _CH
_NA = 10240
_ZR = 80
_ZZ = 40
_HZ = 40
_RPT = _NA // _NS
_B = 1000
_GRID = _N // _B

_SELU_L = 1.0507009873554805
_SELU_A = 1.6732632423543772


def _selu(v):
    return _SELU_L * jnp.where(v > 0, v, _SELU_A * (jnp.exp(jnp.minimum(v, 0.0)) - 1.0))


def _fill(ref, nrows, ncols, value):
    @pl.loop(0, nrows)
    def _row(r):
        for c4 in range(ncols // 16):
            ref[r, pl.ds(c4 * 16, 16)] = jnp.full((16,), value, jnp.float32)


def _zero_acc(acc, zbuf, s, sem0, sem1):
    nb = _RPT // _ZZ
    sems = (sem0, sem1)
    for b in range(nb):
        if b >= 2:
            pltpu.make_async_copy(
                zbuf, acc.at[pl.ds(s * _RPT + (b - 2) * _ZZ, _ZZ)],
                sems[b % 2]).wait()
        pltpu.async_copy(zbuf, acc.at[pl.ds(s * _RPT + b * _ZZ, _ZZ)], sems[b % 2])
    for b in range(nb - 2, nb):
        pltpu.make_async_copy(zbuf, acc.at[pl.ds(s * _RPT + b * _ZZ, _ZZ)],
                              sems[b % 2]).wait()


def _acc_out(acc, s, yref, buf0, buf1, sem0, sem1):
    nb = _RPT // _CH
    bufs, sems = (buf0, buf1), (sem0, sem1)

    def _off(b):
        return s * _RPT + b * _CH

    pltpu.async_copy(acc.at[pl.ds(_off(0), _CH)], buf0, sem0)
    for b in range(nb):
        cur, sem = bufs[b % 2], sems[b % 2]
        pltpu.make_async_copy(acc.at[pl.ds(_off(b), _CH)], cur, sem).wait()
        if b + 1 < nb:
            pltpu.async_copy(acc.at[pl.ds(_off(b + 1), _CH)],
                             bufs[(b + 1) % 2], sems[(b + 1) % 2])
        pltpu.sync_copy(cur, yref.at[pl.ds(_off(b), _CH)])


def _mv_body(table_hbm, p_hbm, y0_hbm, y1_hbm,
             acc, zbuf, pall, sidx0, didx0, sidx1, didx1,
             rows0, rows1, gs0, gs1):
    c = lax.axis_index("c")
    s = lax.axis_index("s")
    w = c * _NS + s

    _fill(zbuf, _ZZ, _D, 0.0)
    pltpu.sync_copy(p_hbm.at[pl.ds(w * _CHUNKS, _CHUNKS)], pall)
    _zero_acc(acc, zbuf, s, gs0, gs1)
    plsc.subcore_barrier()

    def _unpack(i, sbuf, dbuf):
        for j in range(_CH // 16):
            v = pall[i, pl.ds(j * 16, 16)]
            sbuf[pl.ds(j * 16, 16)] = lax.shift_right_logical(v, 14)
            dbuf[pl.ds(j * 16, 16)] = lax.bitwise_and(v, 16383)

    _unpack(0, sidx0, didx0)
    pltpu.async_copy(table_hbm.at[sidx0], rows0, gs0)

    @pl.loop(0, _CHUNKS, step=2)
    def _chunk(i):
        _unpack(i + 1, sidx1, didx1)
        pltpu.make_async_copy(table_hbm.at[sidx0], rows0, gs0).wait()
        pltpu.async_copy(table_hbm.at[sidx1], rows1, gs1)
        pltpu.sync_copy(rows0, acc.at[didx0], add=True)

        @pl.when(i + 2 < _CHUNKS)
        def _():
            _unpack(i + 2, sidx0, didx0)
            pltpu.async_copy(table_hbm.at[sidx0], rows0, gs0)

        pltpu.make_async_copy(table_hbm.at[sidx1], rows1, gs1).wait()
        pltpu.sync_copy(rows1, acc.at[didx1], add=True)

    plsc.subcore_barrier()

    @pl.when(c == 0)
    def _():
        _acc_out(acc, s, y0_hbm, rows0, rows1, gs0, gs1)

    @pl.when(c == 1)
    def _():
        _acc_out(acc, s, y1_hbm, rows0, rows1, gs0, gs1)


def _mv(table, p2):
    mesh = plsc.VectorSubcoreMesh(core_axis_name="c", subcore_axis_name="s")
    f = pl.kernel(
        _mv_body,
        out_type=(jax.ShapeDtypeStruct((_NA, _D), jnp.float32),
                  jax.ShapeDtypeStruct((_NA, _D), jnp.float32)),
        mesh=mesh,
        scratch_types=[
            pltpu.VMEM_SHARED((_NA, _D), jnp.float32),
            pltpu.VMEM((_ZZ, _D), jnp.float32),
            pltpu.VMEM((_CHUNKS, _CH), jnp.int32),
            pltpu.VMEM((_CH,), jnp.int32),
            pltpu.VMEM((_CH,), jnp.int32),
            pltpu.VMEM((_CH,), jnp.int32),
            pltpu.VMEM((_CH,), jnp.int32),
            pltpu.VMEM((_CH, _D), jnp.float32),
            pltpu.VMEM((_CH, _D), jnp.float32),
            pltpu.SemaphoreType.DMA,
            pltpu.SemaphoreType.DMA,
        ],
    )
    return f(table, p2)


def _hist_body(s_hbm, d0_hbm, d1_hbm, acc, zbuf, ones, sall):
    c = lax.axis_index("c")
    s = lax.axis_index("s")
    w = c * _NS + s

    _fill(zbuf, _HZ, _D, 0.0)
    _fill(ones, _CH, _D, 1.0)

    @pl.loop(0, _RPT // _HZ)
    def _z(b):
        pltpu.sync_copy(zbuf, acc.at[pl.ds(s * _RPT + b * _HZ, _HZ)])

    pltpu.sync_copy(s_hbm.at[pl.ds(w * _CHUNKS, _CHUNKS)], sall)
    plsc.subcore_barrier()

    @pl.loop(0, _CHUNKS)
    def _chunk(i):
        pltpu.sync_copy(ones, acc.at[sall.at[i]], add=True)

    plsc.subcore_barrier()

    def _wout(dref):
        for b in range(_RPT // _CH):
            off = s * _RPT + b * _CH
            pltpu.sync_copy(acc.at[pl.ds(off, _CH)], ones)
            pltpu.sync_copy(ones, dref.at[pl.ds(off, _CH)])

    @pl.when(c == 0)
    def _():
        _wout(d0_hbm)

    @pl.when(c == 1)
    def _():
        _wout(d1_hbm)


def _hist(s2):
    mesh = plsc.VectorSubcoreMesh(core_axis_name="c", subcore_axis_name="s")
    f = pl.kernel(
        _hist_body,
        out_type=(jax.ShapeDtypeStruct((_NA, _D), jnp.float32),
                  jax.ShapeDtypeStruct((_NA, _D), jnp.float32)),
        mesh=mesh,
        scratch_types=[
            pltpu.VMEM_SHARED((_NA, _D), jnp.float32),
            pltpu.VMEM((_HZ, _D), jnp.float32),
            pltpu.VMEM((_CH, _D), jnp.float32),
            pltpu.VMEM((_CHUNKS, _CH), jnp.int32),
        ],
    )
    return f(s2)


def _row_spec(w):
    return pl.BlockSpec((_B, w), lambda i: (i, 0))


def _full_spec(shape):
    return pl.BlockSpec(shape, lambda i: (0, 0))


def _prep_kernel(x_ref, d0_ref, d1_ref, w0_ref, scale_ref, xs_ref, o_ref):
    deg = d0_ref[:, 0:1] + d1_ref[:, 0:1]
    sc = jnp.where(deg > 0, -1.0 / deg, 0.0)
    scale_ref[...] = sc
    xs_ref[...] = x_ref[...] * sc
    o_ref[...] = jnp.dot(x_ref[...], w0_ref[...], preferred_element_type=jnp.float32)


def _prep(x, d0, d1, w0):
    return pl.pallas_call(
        _prep_kernel,
        grid=(_GRID,),
        in_specs=[_row_spec(_D), _row_spec(_D), _row_spec(_D), _full_spec((_D, _D))],
        out_specs=[_row_spec(1), _row_spec(_D), _row_spec(_D)],
        out_shape=[jax.ShapeDtypeStruct((_N, 1), jnp.float32),
                   jax.ShapeDtypeStruct((_N, _D), jnp.float32),
                   jax.ShapeDtypeStruct((_N, _D), jnp.float32)],
    )(x, d0, d1, w0)


def _comb_first_kernel(y0_ref, y1_ref, scale_ref, t_ref, xs_ref):
    t = y0_ref[...] + y1_ref[...]
    t_ref[...] = t
    xs_ref[...] = t * scale_ref[...]


def _comb_first(y0, y1, scale):
    return pl.pallas_call(
        _comb_first_kernel,
        grid=(_GRID,),
        in_specs=[_row_spec(_D), _row_spec(_D), _row_spec(1)],
        out_specs=[_row_spec(_D), _row_spec(_D)],
        out_shape=[jax.ShapeDtypeStruct((_N, _D), jnp.float32),
                   jax.ShapeDtypeStruct((_N, _D), jnp.float32)],
    )(y0, y1, scale)


def _comb_mid_kernel(y0_ref, y1_ref, tpp_ref, scale_ref, t_ref, xs_ref):
    t = 2.0 * (y0_ref[...] + y1_ref[...]) - tpp_ref[...]
    t_ref[...] = t
    xs_ref[...] = t * scale_ref[...]


def _comb_mid(y0, y1, tpp, scale):
    return pl.pallas_call(
        _comb_mid_kernel,
        grid=(_GRID,),
        in_specs=[_row_spec(_D), _row_spec(_D), _row_spec(_D), _row_spec(1)],
        out_specs=[_row_spec(_D), _row_spec(_D)],
        out_shape=[jax.ShapeDtypeStruct((_N, _D), jnp.float32),
                   jax.ShapeDtypeStruct((_N, _D), jnp.float32)],
    )(y0, y1, tpp, scale)


def _accmm_kernel(t_ref, o_ref, w_ref, oo_ref):
    oo_ref[...] = o_ref[...] + jnp.dot(t_ref[...], w_ref[...],
                                       preferred_element_type=jnp.float32)


def _accmm(t, o, w):
    return pl.pallas_call(
        _accmm_kernel,
        grid=(_GRID,),
        in_specs=[_row_spec(_D), _row_spec(_D), _full_spec((_D, _D))],
        out_specs=_row_spec(_D),
        out_shape=jax.ShapeDtypeStruct((_N, _D), jnp.float32),
    )(t, o, w)


def _step_last_kernel(y0_ref, y1_ref, tpp_ref, o_ref, w_ref, oo_ref):
    t = 2.0 * (y0_ref[...] + y1_ref[...]) - tpp_ref[...]
    oo_ref[...] = o_ref[...] + jnp.dot(t, w_ref[...], preferred_element_type=jnp.float32)


def _step_last(y0, y1, tpp, o, w):
    return pl.pallas_call(
        _step_last_kernel,
        grid=(_GRID,),
        in_specs=[_row_spec(_D), _row_spec(_D), _row_spec(_D), _row_spec(_D),
                  _full_spec((_D, _D))],
        out_specs=_row_spec(_D),
        out_shape=jax.ShapeDtypeStruct((_N, _D), jnp.float32),
    )(y0, y1, tpp, o, w)


def _bridge_kernel(o_ref, b_ref, w_ref, scale_ref, h_ref, o2_ref, xs_ref):
    h = _selu(o_ref[...] + b_ref[...])
    h_ref[...] = h
    o2_ref[...] = jnp.dot(h, w_ref[...], preferred_element_type=jnp.float32)
    xs_ref[...] = h * scale_ref[...]


def _bridge(o, b1r, w20, scale):
    return pl.pallas_call(
        _bridge_kernel,
        grid=(_GRID,),
        in_specs=[_row_spec(_D), _full_spec((1, _D)), _full_spec((_D, _D)),
                  _row_spec(1)],
        out_specs=[_row_spec(_D), _row_spec(_D), _row_spec(_D)],
        out_shape=[jax.ShapeDtypeStruct((_N, _D), jnp.float32),
                   jax.ShapeDtypeStruct((_N, _D), jnp.float32),
                   jax.ShapeDtypeStruct((_N, _D), jnp.float32)],
    )(o, b1r, w20, scale)


def _final_kernel(y0_ref, y1_ref, tpp_ref, o_ref, w_ref, b_ref, batch_ref,
                  wfc_ref, bfc_ref, out_ref, pooled):
    i = pl.program_id(0)
    t = 2.0 * (y0_ref[...] + y1_ref[...]) - tpp_ref[...]
    o = o_ref[...] + jnp.dot(t, w_ref[...], preferred_element_type=jnp.float32)
    h = _selu(o + b_ref[...])
    bids = batch_ref[0, 0, :]
    onehot = (bids[:, None] == lax.broadcasted_iota(jnp.int32, (1, _G), 1)
              ).astype(jnp.float32)
    p = lax.dot_general(onehot, h, (((0,), (0,)), ((), ())),
                        preferred_element_type=jnp.float32)

    @pl.when(i == 0)
    def _():
        pooled[...] = p

    @pl.when(i > 0)
    def _():
        pooled[...] += p

    @pl.when(i == _GRID - 1)
    def _():
        out_ref[...] = jnp.dot(pooled[...], wfc_ref[...],
                               preferred_element_type=jnp.float32) + bfc_ref[...]


def _final(y0, y1, tpp, o, w, b2r, batch3, wfc, bfcr):
    return pl.pallas_call(
        _final_kernel,
        grid=(_GRID,),
        in_specs=[_row_spec(_D), _row_spec(_D), _row_spec(_D), _row_spec(_D),
                  _full_spec((_D, _D)), _full_spec((1, _D)),
                  pl.BlockSpec((1, 1, _B), lambda i: (i, 0, 0)),
                  _full_spec((_D, _G)), _full_spec((1, _G))],
        out_specs=_full_spec((_G, _G)),
        out_shape=jax.ShapeDtypeStruct((_G, _G), jnp.float32),
        scratch_shapes=[pltpu.VMEM((_G, _D), jnp.float32)],
    )(y0, y1, tpp, o, w, b2r, batch3, wfc, bfcr)


def kernel(x, edge_index, batch, W1, b1, W2, b2, Wfc, bfc):
    src, dst = edge_index[0], edge_index[1]

    pad = _EPAD - _E
    ap = jnp.arange(pad, dtype=jnp.int32)
    gather_pad = (ap * 97) % _N
    sink_pad = _N + (ap % 16)
    gp = jnp.concatenate([src, gather_pad])
    sp = jnp.concatenate([dst, sink_pad])
    p2 = ((gp << 14) | sp).reshape(_EPAD // _CH, _CH)
    hs2 = jnp.concatenate([src, sink_pad]).reshape(_EPAD // _CH, _CH)

    b1r = b1.reshape(1, _D)
    b2r = b2.reshape(1, _D)
    bfcr = bfc.reshape(1, _G)
    batch3 = batch.reshape(_GRID, 1, _B)

    d0, d1 = _hist(hs2)
    scale, xs, o = _prep(x, d0, d1, W1[0])

    tpp = x
    y0, y1 = _mv(xs, p2)
    t, xs = _comb_first(y0, y1, scale)
    o = _accmm(t, o, W1[1])
    for k in (2, 3):
        y0, y1 = _mv(xs, p2)
        tn, xs = _comb_mid(y0, y1, tpp, scale)
        o = _accmm(tn, o, W1[k])
        tpp, t = t, tn
    y0, y1 = _mv(xs, p2)
    o = _step_last(y0, y1, tpp, o, W1[4])

    h, o, xs = _bridge(o, b1r, W2[0], scale)
    tpp = h
    y0, y1 = _mv(xs, p2)
    t, xs = _comb_first(y0, y1, scale)
    o = _accmm(t, o, W2[1])
    for k in (2, 3):
        y0, y1 = _mv(xs, p2)
        tn, xs = _comb_mid(y0, y1, tpp, scale)
        o = _accmm(tn, o, W2[k])
        tpp, t = t, tn
    y0, y1 = _mv(xs, p2)
    return _final(y0, y1, tpp, o, W2[4], b2r, batch3, Wfc, bfcr)


# speedup vs baseline: 13.0365x; 1.0013x over previous
import jax
import jax.numpy as jnp
from jax import lax
from jax.experimental import pallas as pl
from jax.experimental.pallas import tpu as pltpu
from jax.experimental.pallas import tpu_sc as plsc

_N = 10000
_E = 320000
_D = 128
_G = 64
_NC, _NS = 2, 16
_CH = 128
_CHUNKS = 80
_EPAD = _CHUNKS * _NC * _NS * _CH
_NA = 10240
_ZR = 80
_ZZ = 40
_HZ = 40
_RPT = _NA // _NS
_B = 1000
_GRID = _N // _B

_SELU_L = 1.0507009873554805
_SELU_A = 1.6732632423543772


def _selu(v):
    return _SELU_L * jnp.where(v > 0, v, _SELU_A * (jnp.exp(jnp.minimum(v, 0.0)) - 1.0))


def _fill(ref, nrows, ncols, value):
    @pl.loop(0, nrows)
    def _row(r):
        for c4 in range(ncols // 16):
            ref[r, pl.ds(c4 * 16, 16)] = jnp.full((16,), value, jnp.float32)


def _zero_acc(acc, zbuf, s, sem0, sem1):
    nb = _RPT // _ZZ
    sems = (sem0, sem1)
    for b in range(nb):
        if b >= 2:
            pltpu.make_async_copy(
                zbuf, acc.at[pl.ds(s * _RPT + (b - 2) * _ZZ, _ZZ)],
                sems[b % 2]).wait()
        pltpu.async_copy(zbuf, acc.at[pl.ds(s * _RPT + b * _ZZ, _ZZ)], sems[b % 2])
    for b in range(nb - 2, nb):
        pltpu.make_async_copy(zbuf, acc.at[pl.ds(s * _RPT + b * _ZZ, _ZZ)],
                              sems[b % 2]).wait()


def _acc_out(acc, s, yref, buf0, buf1, sem0, sem1):
    nb = _RPT // _CH
    bufs, sems = (buf0, buf1), (sem0, sem1)

    def _off(b):
        return s * _RPT + b * _CH

    pltpu.async_copy(acc.at[pl.ds(_off(0), _CH)], buf0, sem0)
    for b in range(nb):
        cur, sem = bufs[b % 2], sems[b % 2]
        pltpu.make_async_copy(acc.at[pl.ds(_off(b), _CH)], cur, sem).wait()
        if b + 1 < nb:
            pltpu.async_copy(acc.at[pl.ds(_off(b + 1), _CH)],
                             bufs[(b + 1) % 2], sems[(b + 1) % 2])
        pltpu.sync_copy(cur, yref.at[pl.ds(_off(b), _CH)])


def _mv_body(table_hbm, p_hbm, y0_hbm, y1_hbm,
             acc, zbuf, pall, sidx0, didx0, sidx1, didx1,
             rows0, rows1, gs0, gs1):
    c = lax.axis_index("c")
    s = lax.axis_index("s")
    w = c * _NS + s

    _fill(zbuf, _ZZ, _D, 0.0)
    pltpu.sync_copy(p_hbm.at[pl.ds(w * _CHUNKS, _CHUNKS)], pall)
    _zero_acc(acc, zbuf, s, gs0, gs1)
    plsc.subcore_barrier()

    def _unpack(i, sbuf, dbuf):
        for j in range(_CH // 16):
            v = pall[i, pl.ds(j * 16, 16)]
            sbuf[pl.ds(j * 16, 16)] = lax.shift_right_logical(v, 14)
            dbuf[pl.ds(j * 16, 16)] = lax.bitwise_and(v, 16383)

    _unpack(0, sidx0, didx0)
    pltpu.async_copy(table_hbm.at[sidx0], rows0, gs0)

    @pl.loop(0, _CHUNKS, step=2)
    def _chunk(i):
        _unpack(i + 1, sidx1, didx1)
        pltpu.make_async_copy(table_hbm.at[sidx0], rows0, gs0).wait()
        pltpu.async_copy(table_hbm.at[sidx1], rows1, gs1)
        pltpu.sync_copy(rows0, acc.at[didx0], add=True)

        @pl.when(i + 2 < _CHUNKS)
        def _():
            _unpack(i + 2, sidx0, didx0)
            pltpu.async_copy(table_hbm.at[sidx0], rows0, gs0)

        pltpu.make_async_copy(table_hbm.at[sidx1], rows1, gs1).wait()
        pltpu.sync_copy(rows1, acc.at[didx1], add=True)

    plsc.subcore_barrier()

    @pl.when(c == 0)
    def _():
        _acc_out(acc, s, y0_hbm, rows0, rows1, gs0, gs1)

    @pl.when(c == 1)
    def _():
        _acc_out(acc, s, y1_hbm, rows0, rows1, gs0, gs1)


def _mv(table, p2):
    mesh = plsc.VectorSubcoreMesh(core_axis_name="c", subcore_axis_name="s")
    f = pl.kernel(
        _mv_body,
        out_type=(jax.ShapeDtypeStruct((_NA, _D), jnp.float32),
                  jax.ShapeDtypeStruct((_NA, _D), jnp.float32)),
        mesh=mesh,
        scratch_types=[
            pltpu.VMEM_SHARED((_NA, _D), jnp.float32),
            pltpu.VMEM((_ZZ, _D), jnp.float32),
            pltpu.VMEM((_CHUNKS, _CH), jnp.int32),
            pltpu.VMEM((_CH,), jnp.int32),
            pltpu.VMEM((_CH,), jnp.int32),
            pltpu.VMEM((_CH,), jnp.int32),
            pltpu.VMEM((_CH,), jnp.int32),
            pltpu.VMEM((_CH, _D), jnp.float32),
            pltpu.VMEM((_CH, _D), jnp.float32),
            pltpu.SemaphoreType.DMA,
            pltpu.SemaphoreType.DMA,
        ],
    )
    return f(table, p2)


def _hist_body(s_hbm, d0_hbm, d1_hbm, acc, zbuf, ones, sall):
    c = lax.axis_index("c")
    s = lax.axis_index("s")
    w = c * _NS + s

    _fill(zbuf, _HZ, _D, 0.0)
    _fill(ones, _CH, _D, 1.0)

    @pl.loop(0, _RPT // _HZ)
    def _z(b):
        pltpu.sync_copy(zbuf, acc.at[pl.ds(s * _RPT + b * _HZ, _HZ)])

    pltpu.sync_copy(s_hbm.at[pl.ds(w * _CHUNKS, _CHUNKS)], sall)
    plsc.subcore_barrier()

    @pl.loop(0, _CHUNKS)
    def _chunk(i):
        pltpu.sync_copy(ones, acc.at[sall.at[i]], add=True)

    plsc.subcore_barrier()

    def _wout(dref):
        for b in range(_RPT // _CH):
            off = s * _RPT + b * _CH
            pltpu.sync_copy(acc.at[pl.ds(off, _CH)], ones)
            pltpu.sync_copy(ones, dref.at[pl.ds(off, _CH)])

    @pl.when(c == 0)
    def _():
        _wout(d0_hbm)

    @pl.when(c == 1)
    def _():
        _wout(d1_hbm)


def _hist(s2):
    mesh = plsc.VectorSubcoreMesh(core_axis_name="c", subcore_axis_name="s")
    f = pl.kernel(
        _hist_body,
        out_type=(jax.ShapeDtypeStruct((_NA, _D), jnp.float32),
                  jax.ShapeDtypeStruct((_NA, _D), jnp.float32)),
        mesh=mesh,
        scratch_types=[
            pltpu.VMEM_SHARED((_NA, _D), jnp.float32),
            pltpu.VMEM((_HZ, _D), jnp.float32),
            pltpu.VMEM((_CH, _D), jnp.float32),
            pltpu.VMEM((_CHUNKS, _CH), jnp.int32),
        ],
    )
    return f(s2)


def _row_spec(w):
    return pl.BlockSpec((_B, w), lambda i: (i, 0))


def _full_spec(shape):
    return pl.BlockSpec(shape, lambda i: (0, 0))


def _prep_kernel(x_ref, d0_ref, d1_ref, scale_ref, xs_ref):
    deg = d0_ref[:, 0:1] + d1_ref[:, 0:1]
    sc = jnp.where(deg > 0, -1.0 / deg, 0.0)
    scale_ref[...] = sc
    xs_ref[...] = x_ref[...] * sc


def _prep(x, d0, d1):
    return pl.pallas_call(
        _prep_kernel,
        grid=(_GRID,),
        in_specs=[_row_spec(_D), _row_spec(_D), _row_spec(_D)],
        out_specs=[_row_spec(1), _row_spec(_D)],
        out_shape=[jax.ShapeDtypeStruct((_N, 1), jnp.float32),
                   jax.ShapeDtypeStruct((_N, _D), jnp.float32)],
    )(x, d0, d1)


def _mm_kernel(t_ref, w_ref, oo_ref):
    oo_ref[...] = jnp.dot(t_ref[...], w_ref[...], preferred_element_type=jnp.float32)


def _mm(t, w):
    return pl.pallas_call(
        _mm_kernel,
        grid=(_GRID,),
        in_specs=[_row_spec(_D), _full_spec((_D, _D))],
        out_specs=_row_spec(_D),
        out_shape=jax.ShapeDtypeStruct((_N, _D), jnp.float32),
    )(t, w)


def _comb_first_kernel(y0_ref, y1_ref, scale_ref, t_ref, xs_ref):
    t = y0_ref[...] + y1_ref[...]
    t_ref[...] = t
    xs_ref[...] = t * scale_ref[...]


def _comb_first(y0, y1, scale):
    return pl.pallas_call(
        _comb_first_kernel,
        grid=(_GRID,),
        in_specs=[_row_spec(_D), _row_spec(_D), _row_spec(1)],
        out_specs=[_row_spec(_D), _row_spec(_D)],
        out_shape=[jax.ShapeDtypeStruct((_N, _D), jnp.float32),
                   jax.ShapeDtypeStruct((_N, _D), jnp.float32)],
    )(y0, y1, scale)


def _comb_mid_kernel(y0_ref, y1_ref, tpp_ref, scale_ref, t_ref, xs_ref):
    t = 2.0 * (y0_ref[...] + y1_ref[...]) - tpp_ref[...]
    t_ref[...] = t
    xs_ref[...] = t * scale_ref[...]


def _comb_mid(y0, y1, tpp, scale):
    return pl.pallas_call(
        _comb_mid_kernel,
        grid=(_GRID,),
        in_specs=[_row_spec(_D), _row_spec(_D), _row_spec(_D), _row_spec(1)],
        out_specs=[_row_spec(_D), _row_spec(_D)],
        out_shape=[jax.ShapeDtypeStruct((_N, _D), jnp.float32),
                   jax.ShapeDtypeStruct((_N, _D), jnp.float32)],
    )(y0, y1, tpp, scale)


def _accmm_kernel(t_ref, o_ref, w_ref, oo_ref):
    oo_ref[...] = o_ref[...] + jnp.dot(t_ref[...], w_ref[...],
                                       preferred_element_type=jnp.float32)


def _accmm(t, o, w):
    return pl.pallas_call(
        _accmm_kernel,
        grid=(_GRID,),
        in_specs=[_row_spec(_D), _row_spec(_D), _full_spec((_D, _D))],
        out_specs=_row_spec(_D),
        out_shape=jax.ShapeDtypeStruct((_N, _D), jnp.float32),
    )(t, o, w)


def _step_last_kernel(y0_ref, y1_ref, tpp_ref, o_ref, w_ref, oo_ref):
    t = 2.0 * (y0_ref[...] + y1_ref[...]) - tpp_ref[...]
    oo_ref[...] = o_ref[...] + jnp.dot(t, w_ref[...], preferred_element_type=jnp.float32)


def _step_last(y0, y1, tpp, o, w):
    return pl.pallas_call(
        _step_last_kernel,
        grid=(_GRID,),
        in_specs=[_row_spec(_D), _row_spec(_D), _row_spec(_D), _row_spec(_D),
                  _full_spec((_D, _D))],
        out_specs=_row_spec(_D),
        out_shape=jax.ShapeDtypeStruct((_N, _D), jnp.float32),
    )(y0, y1, tpp, o, w)


def _bridge_kernel(o_ref, b_ref, scale_ref, h_ref, xs_ref):
    h = _selu(o_ref[...] + b_ref[...])
    h_ref[...] = h
    xs_ref[...] = h * scale_ref[...]


def _bridge(o, b1r, scale):
    return pl.pallas_call(
        _bridge_kernel,
        grid=(_GRID,),
        in_specs=[_row_spec(_D), _full_spec((1, _D)), _row_spec(1)],
        out_specs=[_row_spec(_D), _row_spec(_D)],
        out_shape=[jax.ShapeDtypeStruct((_N, _D), jnp.float32),
                   jax.ShapeDtypeStruct((_N, _D), jnp.float32)],
    )(o, b1r, scale)


def _final_kernel(y0_ref, y1_ref, tpp_ref, o_ref, w_ref, b_ref, batch_ref,
                  wfc_ref, bfc_ref, out_ref, pooled):
    i = pl.program_id(0)
    t = 2.0 * (y0_ref[...] + y1_ref[...]) - tpp_ref[...]
    o = o_ref[...] + jnp.dot(t, w_ref[...], preferred_element_type=jnp.float32)
    h = _selu(o + b_ref[...])
    bids = batch_ref[0, 0, :]
    onehot = (bids[:, None] == lax.broadcasted_iota(jnp.int32, (1, _G), 1)
              ).astype(jnp.float32)
    p = lax.dot_general(onehot, h, (((0,), (0,)), ((), ())),
                        preferred_element_type=jnp.float32)

    @pl.when(i == 0)
    def _():
        pooled[...] = p

    @pl.when(i > 0)
    def _():
        pooled[...] += p

    @pl.when(i == _GRID - 1)
    def _():
        out_ref[...] = jnp.dot(pooled[...], wfc_ref[...],
                               preferred_element_type=jnp.float32) + bfc_ref[...]


def _final(y0, y1, tpp, o, w, b2r, batch3, wfc, bfcr):
    return pl.pallas_call(
        _final_kernel,
        grid=(_GRID,),
        in_specs=[_row_spec(_D), _row_spec(_D), _row_spec(_D), _row_spec(_D),
                  _full_spec((_D, _D)), _full_spec((1, _D)),
                  pl.BlockSpec((1, 1, _B), lambda i: (i, 0, 0)),
                  _full_spec((_D, _G)), _full_spec((1, _G))],
        out_specs=_full_spec((_G, _G)),
        out_shape=jax.ShapeDtypeStruct((_G, _G), jnp.float32),
        scratch_shapes=[pltpu.VMEM((_G, _D), jnp.float32)],
    )(y0, y1, tpp, o, w, b2r, batch3, wfc, bfcr)


def kernel(x, edge_index, batch, W1, b1, W2, b2, Wfc, bfc):
    src, dst = edge_index[0], edge_index[1]

    pad = _EPAD - _E
    ap = jnp.arange(pad, dtype=jnp.int32)
    gather_pad = (ap * 97) % _N
    sink_pad = _N + (ap % 16)
    gp = jnp.concatenate([src, gather_pad])
    sp = jnp.concatenate([dst, sink_pad])
    p2 = ((gp << 14) | sp).reshape(_EPAD // _CH, _CH)
    hs2 = jnp.concatenate([src, sink_pad]).reshape(_EPAD // _CH, _CH)

    b1r = b1.reshape(1, _D)
    b2r = b2.reshape(1, _D)
    bfcr = bfc.reshape(1, _G)
    batch3 = batch.reshape(_GRID, 1, _B)

    d0, d1 = _hist(hs2)
    scale, xs = _prep(x, d0, d1)
    o = _mm(x, W1[0])

    tpp = x
    y0, y1 = _mv(xs, p2)
    t, xs = _comb_first(y0, y1, scale)
    o = _accmm(t, o, W1[1])
    for k in (2, 3):
        y0, y1 = _mv(xs, p2)
        tn, xs = _comb_mid(y0, y1, tpp, scale)
        o = _accmm(tn, o, W1[k])
        tpp, t = t, tn
    y0, y1 = _mv(xs, p2)
    o = _step_last(y0, y1, tpp, o, W1[4])

    h, xs = _bridge(o, b1r, scale)
    o = _mm(h, W2[0])
    tpp = h
    y0, y1 = _mv(xs, p2)
    t, xs = _comb_first(y0, y1, scale)
    o = _accmm(t, o, W2[1])
    for k in (2, 3):
        y0, y1 = _mv(xs, p2)
        tn, xs = _comb_mid(y0, y1, tpp, scale)
        o = _accmm(tn, o, W2[k])
        tpp, t = t, tn
    y0, y1 = _mv(xs, p2)
    return _final(y0, y1, tpp, o, W2[4], b2r, batch3, Wfc, bfcr)
